# baseline (speedup 1.0000x reference)
import functools

import jax
import jax.numpy as jnp
from jax import lax
from jax.experimental import pallas as pl
from jax.experimental.pallas import tpu as pltpu
from jax.experimental.pallas import tpu_sc as plsc

_NC_SC = 2
_NS_SC = 16
_NW = _NC_SC * _NS_SC
_LANES = 16
_CHUNK = 128


def _linear(x, W, b, act_relu=False, block_rows=512):
    M, K = x.shape
    F = W.shape[-1]

    def body(x_ref, w_ref, b_ref, o_ref):
        y = jnp.dot(x_ref[...], w_ref[...], preferred_element_type=jnp.float32)
        y = y + b_ref[...]
        if act_relu:
            y = jnp.maximum(y, 0.0)
        o_ref[...] = y

    return pl.pallas_call(
        body,
        grid=(pl.cdiv(M, block_rows),),
        in_specs=[
            pl.BlockSpec((block_rows, K), lambda i: (i, 0)),
            pl.BlockSpec((K, F), lambda i: (0, 0)),
            pl.BlockSpec((1, F), lambda i: (0, 0)),
        ],
        out_specs=pl.BlockSpec((block_rows, F), lambda i: (i, 0)),
        out_shape=jax.ShapeDtypeStruct((M, F), jnp.float32),
    )(x, W, b.reshape(1, -1))


def _edge_encode(edge_attr, W, b):
    E, DE = edge_attr.shape
    D = W.shape[1]
    P = 8
    x8 = edge_attr.reshape(E // P, P * DE)
    Wb = jnp.zeros((P * DE, P * D), jnp.float32)
    for i in range(P):
        Wb = Wb.at[i * DE:(i + 1) * DE, i * D:(i + 1) * D].set(W)
    bb = jnp.tile(b, P)
    return _linear(x8, Wb, bb, block_rows=512).reshape(E, D)


def _gine_update(h, agg2, W1, b1, W2, b2, residual):
    N, D = h.shape
    BR = 1000

    def body(h_ref, a_ref, w1_ref, b1_ref, w2_ref, b2_ref, o_ref):
        t = h_ref[...] + a_ref[0] + a_ref[1]
        y = jnp.dot(t, w1_ref[...], preferred_element_type=jnp.float32) + b1_ref[...]
        y = jnp.maximum(y, 0.0)
        y = jnp.dot(y, w2_ref[...], preferred_element_type=jnp.float32) + b2_ref[...]
        y = jnp.maximum(y, 0.0)
        if residual:
            y = h_ref[...] + y
        o_ref[...] = y

    return pl.pallas_call(
        body,
        grid=(pl.cdiv(N, BR),),
        in_specs=[
            pl.BlockSpec((BR, D), lambda i: (i, 0)),
            pl.BlockSpec((2, BR, D), lambda i: (0, i, 0)),
            pl.BlockSpec((D, D), lambda i: (0, 0)),
            pl.BlockSpec((1, D), lambda i: (0, 0)),
            pl.BlockSpec((D, D), lambda i: (0, 0)),
            pl.BlockSpec((1, D), lambda i: (0, 0)),
        ],
        out_specs=pl.BlockSpec((BR, D), lambda i: (i, 0)),
        out_shape=jax.ShapeDtypeStruct((N, D), jnp.float32),
    )(h, agg2, W1, b1.reshape(1, -1), W2, b2.reshape(1, -1))


def _pool_head(h, W1, b1, W2, b2):
    N, D = h.shape
    NC = W2.shape[-1]
    BR = 1000
    NB = N // BR

    def body(h_ref, w1_ref, b1_ref, w2_ref, b2_ref, o_ref, acc):
        i = pl.program_id(0)

        @pl.when(i == 0)
        def _():
            acc[...] = jnp.zeros_like(acc)

        acc[...] += jnp.sum(h_ref[...], axis=0, keepdims=True)

        @pl.when(i == NB - 1)
        def _():
            g = acc[...] * (1.0 / N)
            y = jnp.dot(g, w1_ref[...], preferred_element_type=jnp.float32)
            y = jnp.maximum(y + b1_ref[...], 0.0)
            o_ref[...] = (
                jnp.dot(y, w2_ref[...], preferred_element_type=jnp.float32)
                + b2_ref[...]
            )

    return pl.pallas_call(
        body,
        grid=(NB,),
        in_specs=[
            pl.BlockSpec((BR, D), lambda i: (i, 0)),
            pl.BlockSpec((D, D), lambda i: (0, 0)),
            pl.BlockSpec((1, D), lambda i: (0, 0)),
            pl.BlockSpec((D, NC), lambda i: (0, 0)),
            pl.BlockSpec((1, NC), lambda i: (0, 0)),
        ],
        out_specs=pl.BlockSpec((1, NC), lambda i: (0, 0)),
        out_shape=jax.ShapeDtypeStruct((1, NC), jnp.float32),
        scratch_shapes=[pltpu.VMEM((1, D), jnp.float32)],
    )(h, W1, b1.reshape(1, -1), W2, b2.reshape(1, -1))


def _score_rows(rA_v, rB_v, w2_v, s16_v, n_rows):
    D = rA_v.shape[1]
    w2c = [w2_v[pl.ds(j * _LANES, _LANES)] for j in range(D // _LANES)]

    def edge_body(i, _):
        acc = jnp.zeros((_LANES,), jnp.float32)
        for j in range(D // _LANES):
            sl = pl.ds(j * _LANES, _LANES)
            acc = acc + jnp.maximum(rA_v[i, sl] + rB_v[i, sl], 0.0) * w2c[j]
        s16_v[i, pl.ds(0, _LANES)] = acc
        return 0

    lax.fori_loop(0, n_rows, edge_body, 0)


def _fold_scores_seg(S, b2d, b2a, split, nblk_split, nblk):
    M = S.shape[0]
    S2 = S.reshape(M // 8, 128)
    BR = 2000
    F = jnp.repeat(jnp.eye(8, dtype=jnp.float32), _LANES, axis=0)

    def body(s_ref, f_ref, b_ref, o_ref):
        i = pl.program_id(0)
        y = jnp.dot(s_ref[...], f_ref[...], preferred_element_type=jnp.float32)
        b2 = jnp.where(i < nblk_split, b_ref[0, 0], b_ref[0, 1])
        o_ref[...] = jax.nn.sigmoid(y + b2)

    out = pl.pallas_call(
        body,
        grid=(pl.cdiv(M // 8, BR),),
        in_specs=[pl.BlockSpec((BR, 128), lambda i: (i, 0)),
                  pl.BlockSpec((128, 8), lambda i: (0, 0)),
                  pl.BlockSpec((1, 2), lambda i: (0, 0))],
        out_specs=pl.BlockSpec((BR, 8), lambda i: (i, 0)),
        out_shape=jax.ShapeDtypeStruct((M // 8, 8), jnp.float32),
    )(S2, F, jnp.stack([b2d, b2a]).reshape(1, 2))
    return out.reshape(M)


def _sc_edge_scores(Ad, Bd, Aa, Ba, ia, ib, W2d, W2a, split):
    N, D = Ad.shape
    M = ia.shape[0]
    K = _CHUNK
    nf = M // K
    r = M - nf * K
    w2d = W2d[:, 0]
    w2a = W2a[:, 0]
    mesh = plsc.VectorSubcoreMesh(core_axis_name="c", subcore_axis_name="s")

    scratch = [
        pltpu.VMEM((2, K), jnp.int32),
        pltpu.VMEM((2, K), jnp.int32),
        pltpu.VMEM((2, K, D), jnp.float32),
        pltpu.VMEM((2, K, D), jnp.float32),
        pltpu.VMEM((2, K, _LANES), jnp.float32),
        pltpu.VMEM((D,), jnp.float32),
        pltpu.VMEM((D,), jnp.float32),
    ] + [pltpu.SemaphoreType.DMA] * 6
    if r:
        scratch += [pltpu.VMEM((r,), jnp.int32), pltpu.VMEM((r,), jnp.int32)]

    @functools.partial(
        pl.kernel, mesh=mesh,
        out_type=jax.ShapeDtypeStruct((M, _LANES), jnp.float32),
        scratch_types=scratch,
    )
    def k(Ad_h, Bd_h, Aa_h, Ba_h, ia_h, ib_h, w2d_h, w2a_h, out_h,
          ia_v, ib_v, rA_v, rB_v, s16_v, w2d_v, w2a_v,
          sem_l0, sem_l1, sem_g0, sem_g1, sem_s0, sem_s1, *rest):
        if True:
            w = lax.axis_index("s") * _NC_SC + lax.axis_index("c")
            pltpu.sync_copy(w2d_h, w2d_v)
            pltpu.sync_copy(w2a_h, w2a_v)
            nt = nf // _NW + jnp.where(w < nf % _NW, 1, 0)
            sem_l = (sem_l0, sem_l1)
            sem_g = (sem_g0, sem_g1)
            sem_s = (sem_s0, sem_s1)
            iav = (ia_v.at[0], ia_v.at[1])
            ibv = (ib_v.at[0], ib_v.at[1])
            rAv = (rA_v.at[0], rA_v.at[1])
            rBv = (rB_v.at[0], rB_v.at[1])
            s16 = (s16_v.at[0], s16_v.at[1])

            def lin_copies(i, b):
                base = (w + i * _NW) * K
                return (
                    pltpu.make_async_copy(ia_h.at[pl.ds(base, K)], iav[b],
                                          sem_l[b]),
                    pltpu.make_async_copy(ib_h.at[pl.ds(base, K)], ibv[b],
                                          sem_l[b]),
                )

            def gather_copies(b, seg_add):
                A_h, B_h = (Aa_h, Ba_h) if seg_add else (Ad_h, Bd_h)
                return (
                    pltpu.make_async_copy(A_h.at[iav[b]], rAv[b], sem_g[b]),
                    pltpu.make_async_copy(B_h.at[ibv[b]], rBv[b], sem_g[b]),
                )

            def gather_do(i, b, action):
                base = (w + i * _NW) * K

                @pl.when(base < split)
                def _():
                    for cp in gather_copies(b, False):
                        getattr(cp, action)()

                @pl.when(base >= split)
                def _():
                    for cp in gather_copies(b, True):
                        getattr(cp, action)()

            def store_copy(i, b):
                base = (w + i * _NW) * K
                return pltpu.make_async_copy(
                    s16[b], out_h.at[pl.ds(base, K)], sem_s[b])

            def issue(copies):
                for cp in copies:
                    cp.start()

            def wait(copies):
                for cp in copies:
                    cp.wait()

            @pl.when(nt > 0)
            def _():
                issue(lin_copies(0, 0))
                wait(lin_copies(0, 0))
                gather_do(0, 0, "start")

                @pl.when(nt > 1)
                def _():
                    issue(lin_copies(1, 1))

            def step(i, b):
                gather_do(i, b, "wait")

                @pl.when(i + 1 < nt)
                def _():
                    wait(lin_copies(i + 1, b ^ 1))
                    gather_do(i + 1, b ^ 1, "start")

                @pl.when(i >= 2)
                def _():
                    wait((store_copy(i - 2, b),))

                base = (w + i * _NW) * K
                lax.cond(base < split,
                         lambda: _score_rows(rAv[b], rBv[b], w2d_v,
                                             s16[b], K),
                         lambda: _score_rows(rAv[b], rBv[b], w2a_v,
                                             s16[b], K))
                issue((store_copy(i, b),))

                @pl.when(i + 2 < nt)
                def _():
                    issue(lin_copies(i + 2, b))

            def pair_body(t, _):
                i0 = 2 * t

                @pl.when(i0 < nt)
                def _():
                    step(i0, 0)

                @pl.when(i0 + 1 < nt)
                def _():
                    step(i0 + 1, 1)

                return 0

            lax.fori_loop(0, (nt + 1) // 2, pair_body, 0)

            even = (nt % 2) == 0

            @pl.when(jnp.logical_and(nt >= 2, even))
            def _():
                wait((store_copy(nt - 2, 0), store_copy(nt - 1, 1)))

            @pl.when(jnp.logical_and(nt >= 2, jnp.logical_not(even)))
            def _():
                wait((store_copy(nt - 2, 1), store_copy(nt - 1, 0)))

            @pl.when(nt == 1)
            def _():
                wait((store_copy(0, 0),))

            if r:
                iar_v, ibr_v = rest
                seg_add = nf * K >= split
                Ar_h, Br_h = (Aa_h, Ba_h) if seg_add else (Ad_h, Bd_h)
                w2r_v = w2a_v if seg_add else w2d_v

                @pl.when(w == _NW - 1)
                def _():
                    base = nf * K
                    pltpu.sync_copy(ia_h.at[pl.ds(base, r)], iar_v)
                    pltpu.sync_copy(ib_h.at[pl.ds(base, r)], ibr_v)
                    pltpu.async_copy(
                        Ar_h.at[iar_v], rAv[0].at[pl.ds(0, r)], sem_g0).wait()
                    pltpu.async_copy(
                        Br_h.at[ibr_v], rBv[0].at[pl.ds(0, r)], sem_g0).wait()
                    _score_rows(rAv[0], rBv[0], w2r_v, s16[0], r)
                    pltpu.sync_copy(s16[0].at[pl.ds(0, r)],
                                    out_h.at[pl.ds(base, r)])

    return k(Ad, Bd, Aa, Ba, ia, ib, w2d, w2a)


def _sc_gine_agg(h, e, src, dst, w, split, e_cand_row):
    N, D = h.shape
    M = src.shape[0]
    K = 64
    nf = M // K
    r = M - nf * K
    rows_base = (N // _NS_SC) // 8 * 8
    rows_last = N - rows_base * (_NS_SC - 1)
    zrows = 64

    def _chunks(nrows):
        out, off = [], 0
        while off < nrows:
            sz = min(zrows, nrows - off)
            out.append((off, sz))
            off += sz
        return out
    mesh = plsc.VectorSubcoreMesh(core_axis_name="c", subcore_axis_name="s")

    scratch = [
        pltpu.VMEM((2, K), jnp.int32),
        pltpu.VMEM((4, K), jnp.int32),
        pltpu.VMEM((2, K), jnp.float32),
        pltpu.VMEM((2, K, D), jnp.float32),
        pltpu.VMEM((2, K, D), jnp.float32),
        pltpu.VMEM((zrows, D), jnp.float32),
        pltpu.VMEM((D,), jnp.float32),
        pltpu.VMEM_SHARED((N, D), jnp.float32),
    ] + [pltpu.SemaphoreType.DMA] * 8
    if r:
        scratch += [pltpu.VMEM((r,), jnp.int32), pltpu.VMEM((r,), jnp.int32)]

    @functools.partial(
        pl.kernel, mesh=mesh,
        out_type=jax.ShapeDtypeStruct((2, N, D), jnp.float32),
        scratch_types=scratch,
    )
    def k(h_h, e_h, src_h, dst_h, w_h, ec_h, out_h,
          src_v, dst_v, w_v, hr_v, er_v, z_v, ec_v, agg_sh,
          sem_l0, sem_l1, sem_g0, sem_g1,
          sem_s0, sem_s1, sem_s2, sem_s3, *rest):
        if True:
            c = lax.axis_index("c")
            s = lax.axis_index("s")
            w_id = s * _NC_SC + c
            pltpu.sync_copy(ec_h, ec_v)

            zvec = jnp.zeros((_LANES,), jnp.float32)
            def zbody(i, _):
                for j in range(D // _LANES):
                    z_v[i, pl.ds(j * _LANES, _LANES)] = zvec
                return 0
            lax.fori_loop(0, zrows, zbody, 0)
            row0 = pl.multiple_of(s * rows_base, 8)

            def zero_rows(nrows):
                for off, sz in _chunks(nrows):
                    pltpu.sync_copy(z_v.at[pl.ds(0, sz)],
                                    agg_sh.at[pl.ds(row0 + off, sz)])

            @pl.when(s < _NS_SC - 1)
            def _():
                zero_rows(rows_base)

            @pl.when(s == _NS_SC - 1)
            def _():
                zero_rows(rows_last)

            plsc.subcore_barrier()

            def compute_msgs(b, n_rows, use_cand):
                ngr = (n_rows + _LANES - 1) // _LANES
                if use_cand:
                    ecs = [ec_v[pl.ds(j * _LANES, _LANES)]
                           for j in range(D // _LANES)]

                def grp_body(g, _):
                    wv = w_v[b, pl.ds(g * _LANES, _LANES)]
                    for l in range(_LANES):
                        i = g * _LANES + l
                        wspl = jnp.broadcast_to(wv[l], (_LANES,))
                        for j in range(D // _LANES):
                            sl = pl.ds(j * _LANES, _LANES)
                            ev = ecs[j] if use_cand else er_v[b, i, sl]
                            hr_v[b, i, sl] = jnp.maximum(hr_v[b, i, sl] + ev,
                                                         0.0) * wspl
                    return 0

                lax.fori_loop(0, ngr, grp_body, 0)

            nt = nf // _NW + jnp.where(w_id < nf % _NW, 1, 0)
            sem_l = (sem_l0, sem_l1)
            sem_g = (sem_g0, sem_g1)
            sem_s = (sem_s0, sem_s1, sem_s2, sem_s3)

            def lin_copies(i, b, d):
                base = (w_id + i * _NW) * K
                return base, (
                    pltpu.make_async_copy(src_h.at[pl.ds(base, K)],
                                          src_v.at[b], sem_l[b]),
                    pltpu.make_async_copy(dst_h.at[pl.ds(base, K)],
                                          dst_v.at[d], sem_l[b]),
                    pltpu.make_async_copy(w_h.at[pl.ds(base, K)],
                                          w_v.at[b], sem_l[b]),
                )

            def e_copy(base, b):
                return pltpu.make_async_copy(e_h.at[pl.ds(base, K)],
                                             er_v.at[b], sem_l[b])

            def issue_lin(i, b, d):
                base, cps = lin_copies(i, b, d)
                for cp in cps:
                    cp.start()

                @pl.when(base < split)
                def _():
                    e_copy(base, b).start()

            def wait_lin(i, b, d):
                base, cps = lin_copies(i, b, d)
                for cp in cps:
                    cp.wait()

                @pl.when(base < split)
                def _():
                    e_copy(base, b).wait()

            def gather_copy(b):
                return pltpu.make_async_copy(h_h.at[src_v.at[b]],
                                             hr_v.at[b], sem_g[b])

            def scatter_copy(b, d):
                return pltpu.make_async_copy(hr_v.at[b],
                                             agg_sh.at[dst_v.at[d]],
                                             sem_s[d])

            @pl.when(nt > 0)
            def _():
                issue_lin(0, 0, 0)
                wait_lin(0, 0, 0)
                gather_copy(0).start()

                @pl.when(nt > 1)
                def _():
                    issue_lin(1, 1, 1)

            def step(i, b, d):
                @pl.when(i >= 1)
                def _():
                    scatter_copy(b ^ 1, (d + 3) % 4).wait()

                gather_copy(b).wait()

                @pl.when(i + 1 < nt)
                def _():
                    wait_lin(i + 1, b ^ 1, (d + 1) % 4)
                    gather_copy(b ^ 1).start()

                base = (w_id + i * _NW) * K
                lax.cond(base >= split,
                         lambda: compute_msgs(b, K, True),
                         lambda: compute_msgs(b, K, False))
                scatter_copy(b, d).start(add=True)

                @pl.when(i + 2 < nt)
                def _():
                    issue_lin(i + 2, b, (d + 2) % 4)

            def quad_body(t, _):
                i0 = 4 * t
                for x in range(4):
                    @pl.when(i0 + x < nt)
                    def _():
                        step(i0 + x, x % 2, x)
                return 0

            lax.fori_loop(0, (nt + 3) // 4, quad_body, 0)

            m = nt % 4
            for mm, (bb, dd) in enumerate(((1, 3), (0, 0), (1, 1), (0, 2))):
                @pl.when(jnp.logical_and(nt > 0, m == mm))
                def _(bb=bb, dd=dd):
                    scatter_copy(bb, dd).wait()

            if r:
                srcr_v, dstr_v = rest

                @pl.when(w_id == _NW - 1)
                def _():
                    base = nf * K
                    pltpu.sync_copy(src_h.at[pl.ds(base, r)], srcr_v)
                    pltpu.sync_copy(dst_h.at[pl.ds(base, r)], dstr_v)
                    pltpu.sync_copy(w_h.at[pl.ds(base, r)],
                                    w_v.at[0, pl.ds(0, r)])
                    pltpu.async_copy(h_h.at[srcr_v],
                                     hr_v.at[0, pl.ds(0, r)], sem_g0).wait()
                    if split < M:
                        compute_msgs(0, r, True)
                    else:
                        pltpu.sync_copy(e_h.at[pl.ds(base, r)],
                                        er_v.at[0, pl.ds(0, r)])
                        compute_msgs(0, r, False)
                    pltpu.sync_copy(hr_v.at[0, pl.ds(0, r)],
                                    agg_sh.at[dstr_v], add=True)

            plsc.subcore_barrier()

            def flush_rows(nrows):
                for off, sz in _chunks(nrows):
                    pltpu.sync_copy(agg_sh.at[pl.ds(row0 + off, sz)],
                                    z_v.at[pl.ds(0, sz)])
                    pltpu.sync_copy(z_v.at[pl.ds(0, sz)],
                                    out_h.at[c, pl.ds(row0 + off, sz)])

            @pl.when(s < _NS_SC - 1)
            def _():
                flush_rows(rows_base)

            @pl.when(s == _NS_SC - 1)
            def _():
                flush_rows(rows_last)

    return k(h, e, src, dst, w, e_cand_row)


def kernel(x, edge_index, edge_attr, edge_weight, edge_candidate, params):
    p = params
    N, D = x.shape
    ei0, ei1 = edge_index[0], edge_index[1]
    c0, c1 = edge_candidate[:, 0], edge_candidate[:, 1]
    src_all = jnp.concatenate([ei0, c0])
    dst_all = jnp.concatenate([ei1, c1])

    h = _linear(x, p['enc_W'], p['enc_b'])

    for lp in p['layers']:
        conv, addp, delp, inter = lp['conv'], lp['add'], lp['del'], lp['inter']

        e1 = _edge_encode(edge_attr, conv['W_ee'], conv['b_ee'])
        agg = _sc_gine_agg(h, e1, ei0, ei1, edge_weight,
                           ei0.shape[0], conv['b_ee'])
        h = _gine_update(h, agg, conv['W1'], conv['b1'], conv['W2'], conv['b2'],
                         residual=True)

        Wcat = jnp.concatenate(
            [addp['W1'][:D], addp['W1'][D:], delp['W1'][:D], delp['W1'][D:]],
            axis=1)
        bcat = jnp.concatenate(
            [jnp.zeros((D,), jnp.float32), addp['b1'],
             jnp.zeros((D,), jnp.float32), delp['b1']])
        AB = _linear(h, Wcat, bcat)
        A_add, B_add = AB[:, :D], AB[:, D:2 * D]
        A_del, B_del = AB[:, 2 * D:3 * D], AB[:, 3 * D:]

        E = ei0.shape[0]
        s16 = _sc_edge_scores(A_del, B_del, A_add, B_add, src_all, dst_all,
                              delp['W2'], addp['W2'], E)
        w_all = _fold_scores_seg(s16, delp['b2'][0], addp['b2'][0], E,
                                 nblk_split=(E // 8) // 2000, nblk=0)

        e2 = _edge_encode(edge_attr, inter['W_ee'], inter['b_ee'])
        agg2 = _sc_gine_agg(h, e2, src_all, dst_all, w_all,
                            ei0.shape[0], inter['b_ee'])
        h = _gine_update(h, agg2, inter['W1'], inter['b1'], inter['W2'],
                         inter['b2'], residual=False)

    return _pool_head(h, p['mlp_W1'], p['mlp_b1'], p['mlp_W2'], p['mlp_b2'])


# speedup vs baseline: 1.0193x; 1.0193x over previous
import functools

import jax
import jax.numpy as jnp
from jax import lax
from jax.experimental import pallas as pl
from jax.experimental.pallas import tpu as pltpu
from jax.experimental.pallas import tpu_sc as plsc

_NC_SC = 2
_NS_SC = 16
_NW = _NC_SC * _NS_SC
_LANES = 16
_CHUNK = 128


def _linear(x, W, b, act_relu=False, block_rows=512):
    M, K = x.shape
    F = W.shape[-1]

    def body(x_ref, w_ref, b_ref, o_ref):
        y = jnp.dot(x_ref[...], w_ref[...], preferred_element_type=jnp.float32)
        y = y + b_ref[...]
        if act_relu:
            y = jnp.maximum(y, 0.0)
        o_ref[...] = y

    return pl.pallas_call(
        body,
        grid=(pl.cdiv(M, block_rows),),
        in_specs=[
            pl.BlockSpec((block_rows, K), lambda i: (i, 0)),
            pl.BlockSpec((K, F), lambda i: (0, 0)),
            pl.BlockSpec((1, F), lambda i: (0, 0)),
        ],
        out_specs=pl.BlockSpec((block_rows, F), lambda i: (i, 0)),
        out_shape=jax.ShapeDtypeStruct((M, F), jnp.float32),
    )(x, W, b.reshape(1, -1))


def _edge_encode(edge_attr, W, b):
    E, DE = edge_attr.shape
    D = W.shape[1]
    P = 8
    x8 = edge_attr.reshape(E // P, P * DE)
    Wb = jnp.zeros((P * DE, P * D), jnp.float32)
    for i in range(P):
        Wb = Wb.at[i * DE:(i + 1) * DE, i * D:(i + 1) * D].set(W)
    bb = jnp.tile(b, P)
    return _linear(x8, Wb, bb, block_rows=512)


def _gine_update(h, agg2, W1, b1, W2, b2, residual):
    N, D = h.shape
    BR = 1000

    def body(h_ref, a_ref, w1_ref, b1_ref, w2_ref, b2_ref, o_ref):
        t = h_ref[...] + a_ref[0] + a_ref[1]
        y = jnp.dot(t, w1_ref[...], preferred_element_type=jnp.float32) + b1_ref[...]
        y = jnp.maximum(y, 0.0)
        y = jnp.dot(y, w2_ref[...], preferred_element_type=jnp.float32) + b2_ref[...]
        y = jnp.maximum(y, 0.0)
        if residual:
            y = h_ref[...] + y
        o_ref[...] = y

    return pl.pallas_call(
        body,
        grid=(pl.cdiv(N, BR),),
        in_specs=[
            pl.BlockSpec((BR, D), lambda i: (i, 0)),
            pl.BlockSpec((2, BR, D), lambda i: (0, i, 0)),
            pl.BlockSpec((D, D), lambda i: (0, 0)),
            pl.BlockSpec((1, D), lambda i: (0, 0)),
            pl.BlockSpec((D, D), lambda i: (0, 0)),
            pl.BlockSpec((1, D), lambda i: (0, 0)),
        ],
        out_specs=pl.BlockSpec((BR, D), lambda i: (i, 0)),
        out_shape=jax.ShapeDtypeStruct((N, D), jnp.float32),
    )(h, agg2, W1, b1.reshape(1, -1), W2, b2.reshape(1, -1))


def _pool_head(h, W1, b1, W2, b2):
    N, D = h.shape
    NC = W2.shape[-1]
    BR = 1000
    NB = N // BR

    def body(h_ref, w1_ref, b1_ref, w2_ref, b2_ref, o_ref, acc):
        i = pl.program_id(0)

        @pl.when(i == 0)
        def _():
            acc[...] = jnp.zeros_like(acc)

        acc[...] += jnp.sum(h_ref[...], axis=0, keepdims=True)

        @pl.when(i == NB - 1)
        def _():
            g = acc[...] * (1.0 / N)
            y = jnp.dot(g, w1_ref[...], preferred_element_type=jnp.float32)
            y = jnp.maximum(y + b1_ref[...], 0.0)
            o_ref[...] = (
                jnp.dot(y, w2_ref[...], preferred_element_type=jnp.float32)
                + b2_ref[...]
            )

    return pl.pallas_call(
        body,
        grid=(NB,),
        in_specs=[
            pl.BlockSpec((BR, D), lambda i: (i, 0)),
            pl.BlockSpec((D, D), lambda i: (0, 0)),
            pl.BlockSpec((1, D), lambda i: (0, 0)),
            pl.BlockSpec((D, NC), lambda i: (0, 0)),
            pl.BlockSpec((1, NC), lambda i: (0, 0)),
        ],
        out_specs=pl.BlockSpec((1, NC), lambda i: (0, 0)),
        out_shape=jax.ShapeDtypeStruct((1, NC), jnp.float32),
        scratch_shapes=[pltpu.VMEM((1, D), jnp.float32)],
    )(h, W1, b1.reshape(1, -1), W2, b2.reshape(1, -1))


def _score_rows(rA_v, rB_v, w2_v, s16_v, n_rows):
    D = rA_v.shape[1]
    w2c = [w2_v[pl.ds(j * _LANES, _LANES)] for j in range(D // _LANES)]
    ngr = (n_rows + _LANES - 1) // _LANES

    def grp_body(g, _):
        for l in range(_LANES):
            i = g * _LANES + l
            acc = jnp.zeros((_LANES,), jnp.float32)
            for j in range(D // _LANES):
                sl = pl.ds(j * _LANES, _LANES)
                acc = acc + jnp.maximum(rA_v[i, sl] + rB_v[i, sl],
                                        0.0) * w2c[j]
            s16_v[2 * g + l // 8, pl.ds((l % 8) * _LANES, _LANES)] = acc
        return 0

    lax.fori_loop(0, ngr, grp_body, 0)


def _fold_scores_seg(S2, b2d, b2a, nblk_split):
    M8 = S2.shape[0]
    BR = 2000
    F = jnp.concatenate(
        [jnp.repeat(jnp.eye(8, dtype=jnp.float32), _LANES, axis=0),
         jnp.zeros((128, 8), jnp.float32)], axis=1)

    def body(s_ref, f_ref, b_ref, o_ref):
        i = pl.program_id(0)
        y = jnp.dot(s_ref[...], f_ref[...], preferred_element_type=jnp.float32)
        b2 = jnp.where(i < nblk_split, b_ref[0, 0], b_ref[0, 1])
        o_ref[...] = jax.nn.sigmoid(y + b2)

    return pl.pallas_call(
        body,
        grid=(pl.cdiv(M8, BR),),
        in_specs=[pl.BlockSpec((BR, 128), lambda i: (i, 0)),
                  pl.BlockSpec((128, _LANES), lambda i: (0, 0)),
                  pl.BlockSpec((1, 2), lambda i: (0, 0))],
        out_specs=pl.BlockSpec((BR, _LANES), lambda i: (i, 0)),
        out_shape=jax.ShapeDtypeStruct((M8, _LANES), jnp.float32),
    )(S2, F, jnp.stack([b2d, b2a]).reshape(1, 2))


def _sc_edge_scores(Ad, Bd, Aa, Ba, ia, ib, W2d, W2a, split):
    N, D = Ad.shape
    M = ia.shape[0]
    K = _CHUNK
    nf = M // K
    r = M - nf * K
    w2d = W2d[:, 0]
    w2a = W2a[:, 0]
    mesh = plsc.VectorSubcoreMesh(core_axis_name="c", subcore_axis_name="s")

    scratch = [
        pltpu.VMEM((2, K), jnp.int32),
        pltpu.VMEM((2, K), jnp.int32),
        pltpu.VMEM((2, K, D), jnp.float32),
        pltpu.VMEM((2, K, D), jnp.float32),
        pltpu.VMEM((2, K // 8, 128), jnp.float32),
        pltpu.VMEM((D,), jnp.float32),
        pltpu.VMEM((D,), jnp.float32),
    ] + [pltpu.SemaphoreType.DMA] * 6
    if r:
        assert r % 8 == 0
        scratch += [pltpu.VMEM((r,), jnp.int32), pltpu.VMEM((r,), jnp.int32)]

    @functools.partial(
        pl.kernel, mesh=mesh,
        out_type=jax.ShapeDtypeStruct((M // 8, 128), jnp.float32),
        scratch_types=scratch,
    )
    def k(Ad_h, Bd_h, Aa_h, Ba_h, ia_h, ib_h, w2d_h, w2a_h, out_h,
          ia_v, ib_v, rA_v, rB_v, s16_v, w2d_v, w2a_v,
          sem_l0, sem_l1, sem_g0, sem_g1, sem_s0, sem_s1, *rest):
        if True:
            w = lax.axis_index("s") * _NC_SC + lax.axis_index("c")
            pltpu.sync_copy(w2d_h, w2d_v)
            pltpu.sync_copy(w2a_h, w2a_v)
            nt = nf // _NW + jnp.where(w < nf % _NW, 1, 0)
            sem_l = (sem_l0, sem_l1)
            sem_g = (sem_g0, sem_g1)
            sem_s = (sem_s0, sem_s1)
            iav = (ia_v.at[0], ia_v.at[1])
            ibv = (ib_v.at[0], ib_v.at[1])
            rAv = (rA_v.at[0], rA_v.at[1])
            rBv = (rB_v.at[0], rB_v.at[1])
            s16 = (s16_v.at[0], s16_v.at[1])

            def lin_copies(i, b):
                base = (w + i * _NW) * K
                return (
                    pltpu.make_async_copy(ia_h.at[pl.ds(base, K)], iav[b],
                                          sem_l[b]),
                    pltpu.make_async_copy(ib_h.at[pl.ds(base, K)], ibv[b],
                                          sem_l[b]),
                )

            def gather_copies(b, seg_add):
                A_h, B_h = (Aa_h, Ba_h) if seg_add else (Ad_h, Bd_h)
                return (
                    pltpu.make_async_copy(A_h.at[iav[b]], rAv[b], sem_g[b]),
                    pltpu.make_async_copy(B_h.at[ibv[b]], rBv[b], sem_g[b]),
                )

            def gather_do(i, b, action):
                base = (w + i * _NW) * K

                @pl.when(base < split)
                def _():
                    for cp in gather_copies(b, False):
                        getattr(cp, action)()

                @pl.when(base >= split)
                def _():
                    for cp in gather_copies(b, True):
                        getattr(cp, action)()

            def store_copy(i, b):
                base8 = (w + i * _NW) * (K // 8)
                return pltpu.make_async_copy(
                    s16[b], out_h.at[pl.ds(base8, K // 8)], sem_s[b])

            def issue(copies):
                for cp in copies:
                    cp.start()

            def wait(copies):
                for cp in copies:
                    cp.wait()

            @pl.when(nt > 0)
            def _():
                issue(lin_copies(0, 0))
                wait(lin_copies(0, 0))
                gather_do(0, 0, "start")

                @pl.when(nt > 1)
                def _():
                    issue(lin_copies(1, 1))

            def step(i, b):
                gather_do(i, b, "wait")

                @pl.when(i + 1 < nt)
                def _():
                    wait(lin_copies(i + 1, b ^ 1))
                    gather_do(i + 1, b ^ 1, "start")

                @pl.when(i >= 2)
                def _():
                    wait((store_copy(i - 2, b),))

                base = (w + i * _NW) * K
                lax.cond(base < split,
                         lambda: _score_rows(rAv[b], rBv[b], w2d_v,
                                             s16[b], K),
                         lambda: _score_rows(rAv[b], rBv[b], w2a_v,
                                             s16[b], K))
                issue((store_copy(i, b),))

                @pl.when(i + 2 < nt)
                def _():
                    issue(lin_copies(i + 2, b))

            def pair_body(t, _):
                i0 = 2 * t

                @pl.when(i0 < nt)
                def _():
                    step(i0, 0)

                @pl.when(i0 + 1 < nt)
                def _():
                    step(i0 + 1, 1)

                return 0

            lax.fori_loop(0, (nt + 1) // 2, pair_body, 0)

            even = (nt % 2) == 0

            @pl.when(jnp.logical_and(nt >= 2, even))
            def _():
                wait((store_copy(nt - 2, 0), store_copy(nt - 1, 1)))

            @pl.when(jnp.logical_and(nt >= 2, jnp.logical_not(even)))
            def _():
                wait((store_copy(nt - 2, 1), store_copy(nt - 1, 0)))

            @pl.when(nt == 1)
            def _():
                wait((store_copy(0, 0),))

            if r:
                iar_v, ibr_v = rest
                seg_add = nf * K >= split
                Ar_h, Br_h = (Aa_h, Ba_h) if seg_add else (Ad_h, Bd_h)
                w2r_v = w2a_v if seg_add else w2d_v

                @pl.when(w == _NW - 1)
                def _():
                    base = nf * K
                    pltpu.sync_copy(ia_h.at[pl.ds(base, r)], iar_v)
                    pltpu.sync_copy(ib_h.at[pl.ds(base, r)], ibr_v)
                    pltpu.async_copy(
                        Ar_h.at[iar_v], rAv[0].at[pl.ds(0, r)], sem_g0).wait()
                    pltpu.async_copy(
                        Br_h.at[ibr_v], rBv[0].at[pl.ds(0, r)], sem_g0).wait()
                    _score_rows(rAv[0], rBv[0], w2r_v, s16[0], r)
                    pltpu.sync_copy(s16[0].at[pl.ds(0, r // 8)],
                                    out_h.at[pl.ds(base // 8, r // 8)])

    return k(Ad, Bd, Aa, Ba, ia, ib, w2d, w2a)


def _sc_gine_agg(h, e, src, dst, w, split, e_cand_row, w_packed):
    N, D = h.shape
    M = src.shape[0]
    K = 64
    nf = M // K
    r = M - nf * K
    rows_base = (N // _NS_SC) // 8 * 8
    rows_last = N - rows_base * (_NS_SC - 1)
    zrows = 64

    def _chunks(nrows):
        out, off = [], 0
        while off < nrows:
            sz = min(zrows, nrows - off)
            out.append((off, sz))
            off += sz
        return out
    mesh = plsc.VectorSubcoreMesh(core_axis_name="c", subcore_axis_name="s")

    w_shape = (2, K // 8, _LANES) if w_packed else (2, K)
    scratch = [
        pltpu.VMEM((2, K), jnp.int32),
        pltpu.VMEM((4, K), jnp.int32),
        pltpu.VMEM(w_shape, jnp.float32),
        pltpu.VMEM((2, K, D), jnp.float32),
        pltpu.VMEM((2, K // 8, 8 * D), jnp.float32),
        pltpu.VMEM((zrows, D), jnp.float32),
        pltpu.VMEM((D,), jnp.float32),
        pltpu.VMEM_SHARED((N, D), jnp.float32),
    ] + [pltpu.SemaphoreType.DMA] * 8
    if r:
        scratch += [pltpu.VMEM((r,), jnp.int32), pltpu.VMEM((r,), jnp.int32)]

    @functools.partial(
        pl.kernel, mesh=mesh,
        out_type=jax.ShapeDtypeStruct((2, N, D), jnp.float32),
        scratch_types=scratch,
    )
    def k(h_h, e_h, src_h, dst_h, w_h, ec_h, out_h,
          src_v, dst_v, w_v, hr_v, er_v, z_v, ec_v, agg_sh,
          sem_l0, sem_l1, sem_g0, sem_g1,
          sem_s0, sem_s1, sem_s2, sem_s3, *rest):
        if True:
            c = lax.axis_index("c")
            s = lax.axis_index("s")
            w_id = s * _NC_SC + c
            pltpu.sync_copy(ec_h, ec_v)

            zvec = jnp.zeros((_LANES,), jnp.float32)
            def zbody(i, _):
                for j in range(D // _LANES):
                    z_v[i, pl.ds(j * _LANES, _LANES)] = zvec
                return 0
            lax.fori_loop(0, zrows, zbody, 0)
            row0 = pl.multiple_of(s * rows_base, 8)

            def zero_rows(nrows):
                for off, sz in _chunks(nrows):
                    pltpu.sync_copy(z_v.at[pl.ds(0, sz)],
                                    agg_sh.at[pl.ds(row0 + off, sz)])

            @pl.when(s < _NS_SC - 1)
            def _():
                zero_rows(rows_base)

            @pl.when(s == _NS_SC - 1)
            def _():
                zero_rows(rows_last)

            plsc.subcore_barrier()

            def compute_msgs(b, n_rows, use_cand):
                ngr = (n_rows + _LANES - 1) // _LANES
                if use_cand:
                    ecs = [ec_v[pl.ds(j * _LANES, _LANES)]
                           for j in range(D // _LANES)]

                def grp_body(g, _):
                    if w_packed:
                        wrow = (w_v[b, 2 * g, pl.ds(0, _LANES)],
                                w_v[b, 2 * g + 1, pl.ds(0, _LANES)])
                    else:
                        wv = w_v[b, pl.ds(g * _LANES, _LANES)]
                    for l in range(_LANES):
                        i = g * _LANES + l
                        if w_packed:
                            wspl = jnp.broadcast_to(wrow[l // 8][l % 8],
                                                    (_LANES,))
                        else:
                            wspl = jnp.broadcast_to(wv[l], (_LANES,))
                        for j in range(D // _LANES):
                            sl = pl.ds(j * _LANES, _LANES)
                            if use_cand:
                                ev = ecs[j]
                            else:
                                ev = er_v[b, 2 * g + l // 8,
                                          pl.ds((l % 8) * D + j * _LANES,
                                                _LANES)]
                            hr_v[b, i, sl] = jnp.maximum(hr_v[b, i, sl] + ev,
                                                         0.0) * wspl
                    return 0

                lax.fori_loop(0, ngr, grp_body, 0)

            nt = nf // _NW + jnp.where(w_id < nf % _NW, 1, 0)
            sem_l = (sem_l0, sem_l1)
            sem_g = (sem_g0, sem_g1)
            sem_s = (sem_s0, sem_s1, sem_s2, sem_s3)

            def lin_copies(i, b, d):
                base = (w_id + i * _NW) * K
                base8 = (w_id + i * _NW) * (K // 8)
                if w_packed:
                    wcp = pltpu.make_async_copy(
                        w_h.at[pl.ds(base8, K // 8)], w_v.at[b], sem_l[b])
                else:
                    wcp = pltpu.make_async_copy(
                        w_h.at[pl.ds(base, K)], w_v.at[b], sem_l[b])
                return base, (
                    pltpu.make_async_copy(src_h.at[pl.ds(base, K)],
                                          src_v.at[b], sem_l[b]),
                    pltpu.make_async_copy(dst_h.at[pl.ds(base, K)],
                                          dst_v.at[d], sem_l[b]),
                    wcp,
                )

            def e_copy(i, b):
                base8 = (w_id + i * _NW) * (K // 8)
                return pltpu.make_async_copy(
                    e_h.at[pl.ds(base8, K // 8)], er_v.at[b], sem_l[b])

            def issue_lin(i, b, d):
                base, cps = lin_copies(i, b, d)
                for cp in cps:
                    cp.start()

                @pl.when(base < split)
                def _():
                    e_copy(i, b).start()

            def wait_lin(i, b, d):
                base, cps = lin_copies(i, b, d)
                for cp in cps:
                    cp.wait()

                @pl.when(base < split)
                def _():
                    e_copy(i, b).wait()

            def gather_copy(b):
                return pltpu.make_async_copy(h_h.at[src_v.at[b]],
                                             hr_v.at[b], sem_g[b])

            def scatter_copy(b, d):
                return pltpu.make_async_copy(hr_v.at[b],
                                             agg_sh.at[dst_v.at[d]],
                                             sem_s[d])

            @pl.when(nt > 0)
            def _():
                issue_lin(0, 0, 0)
                wait_lin(0, 0, 0)
                gather_copy(0).start()

                @pl.when(nt > 1)
                def _():
                    issue_lin(1, 1, 1)

            def step(i, b, d):
                @pl.when(i >= 1)
                def _():
                    scatter_copy(b ^ 1, (d + 3) % 4).wait()

                gather_copy(b).wait()

                @pl.when(i + 1 < nt)
                def _():
                    wait_lin(i + 1, b ^ 1, (d + 1) % 4)
                    gather_copy(b ^ 1).start()

                base = (w_id + i * _NW) * K
                lax.cond(base >= split,
                         lambda: compute_msgs(b, K, True),
                         lambda: compute_msgs(b, K, False))
                scatter_copy(b, d).start(add=True)

                @pl.when(i + 2 < nt)
                def _():
                    issue_lin(i + 2, b, (d + 2) % 4)

            def quad_body(t, _):
                i0 = 4 * t
                for x in range(4):
                    @pl.when(i0 + x < nt)
                    def _():
                        step(i0 + x, x % 2, x)
                return 0

            lax.fori_loop(0, (nt + 3) // 4, quad_body, 0)

            m = nt % 4
            for mm, (bb, dd) in enumerate(((1, 3), (0, 0), (1, 1), (0, 2))):
                @pl.when(jnp.logical_and(nt > 0, m == mm))
                def _(bb=bb, dd=dd):
                    scatter_copy(bb, dd).wait()

            if r:
                srcr_v, dstr_v = rest

                @pl.when(w_id == _NW - 1)
                def _():
                    base = nf * K
                    base8 = nf * (K // 8)
                    pltpu.sync_copy(src_h.at[pl.ds(base, r)], srcr_v)
                    pltpu.sync_copy(dst_h.at[pl.ds(base, r)], dstr_v)
                    if w_packed:
                        pltpu.sync_copy(w_h.at[pl.ds(base8, r // 8)],
                                        w_v.at[0, pl.ds(0, r // 8)])
                    else:
                        pltpu.sync_copy(w_h.at[pl.ds(base, r)],
                                        w_v.at[0, pl.ds(0, r)])
                    pltpu.async_copy(h_h.at[srcr_v],
                                     hr_v.at[0, pl.ds(0, r)], sem_g0).wait()
                    if split < M:
                        compute_msgs(0, r, True)
                    else:
                        pltpu.sync_copy(e_h.at[pl.ds(base8, r // 8)],
                                        er_v.at[0, pl.ds(0, r // 8)])
                        compute_msgs(0, r, False)
                    pltpu.sync_copy(hr_v.at[0, pl.ds(0, r)],
                                    agg_sh.at[dstr_v], add=True)

            plsc.subcore_barrier()

            def flush_rows(nrows):
                for off, sz in _chunks(nrows):
                    pltpu.sync_copy(agg_sh.at[pl.ds(row0 + off, sz)],
                                    z_v.at[pl.ds(0, sz)])
                    pltpu.sync_copy(z_v.at[pl.ds(0, sz)],
                                    out_h.at[c, pl.ds(row0 + off, sz)])

            @pl.when(s < _NS_SC - 1)
            def _():
                flush_rows(rows_base)

            @pl.when(s == _NS_SC - 1)
            def _():
                flush_rows(rows_last)

    return k(h, e, src, dst, w, e_cand_row)


def kernel(x, edge_index, edge_attr, edge_weight, edge_candidate, params):
    p = params
    N, D = x.shape
    ei0, ei1 = edge_index[0], edge_index[1]
    c0, c1 = edge_candidate[:, 0], edge_candidate[:, 1]
    src_all = jnp.concatenate([ei0, c0])
    dst_all = jnp.concatenate([ei1, c1])

    h = _linear(x, p['enc_W'], p['enc_b'])

    for lp in p['layers']:
        conv, addp, delp, inter = lp['conv'], lp['add'], lp['del'], lp['inter']

        e1 = _edge_encode(edge_attr, conv['W_ee'], conv['b_ee'])
        agg = _sc_gine_agg(h, e1, ei0, ei1, edge_weight,
                           ei0.shape[0], conv['b_ee'], w_packed=False)
        h = _gine_update(h, agg, conv['W1'], conv['b1'], conv['W2'], conv['b2'],
                         residual=True)

        Wcat = jnp.concatenate(
            [addp['W1'][:D], addp['W1'][D:], delp['W1'][:D], delp['W1'][D:]],
            axis=1)
        bcat = jnp.concatenate(
            [jnp.zeros((D,), jnp.float32), addp['b1'],
             jnp.zeros((D,), jnp.float32), delp['b1']])
        AB = _linear(h, Wcat, bcat)
        A_add, B_add = AB[:, :D], AB[:, D:2 * D]
        A_del, B_del = AB[:, 2 * D:3 * D], AB[:, 3 * D:]

        E = ei0.shape[0]
        s16 = _sc_edge_scores(A_del, B_del, A_add, B_add, src_all, dst_all,
                              delp['W2'], addp['W2'], E)
        w_all = _fold_scores_seg(s16, delp['b2'][0], addp['b2'][0],
                                 nblk_split=(E // 8) // 2000)

        e2 = _edge_encode(edge_attr, inter['W_ee'], inter['b_ee'])
        agg2 = _sc_gine_agg(h, e2, src_all, dst_all, w_all,
                            ei0.shape[0], inter['b_ee'], w_packed=True)
        h = _gine_update(h, agg2, inter['W1'], inter['b1'], inter['W2'],
                         inter['b2'], residual=False)

    return _pool_head(h, p['mlp_W1'], p['mlp_b1'], p['mlp_W2'], p['mlp_b2'])
